# baseline (speedup 1.0000x reference)
import jax
import jax.numpy as jnp
from jax.experimental import pallas as pl

B = 2
N = 1000
T = 16
F_IN = 64
H = 64
C = 12
R = 1000


def _sigmoid(v):
    return 0.5 * jnp.tanh(0.5 * v) + 0.5


def _split(v):
    hi = v.astype(jnp.bfloat16)
    lo = (v - hi.astype(jnp.float32)).astype(jnp.bfloat16)
    return hi, lo


def _dot3(a_hi, a_lo, b_hi, b_lo):
    f32 = jnp.float32
    return (jnp.dot(a_hi, b_hi, preferred_element_type=f32)
            + jnp.dot(a_lo, b_hi, preferred_element_type=f32)
            + jnp.dot(a_hi, b_lo, preferred_element_type=f32))


def _gru_kernel(xct_ref, wih_ref, whh_ref, bih_ref, bhh_ref, out_ref):
    xbt = xct_ref[...]
    xrev = jnp.concatenate([xbt[T - 1 - t:T - t] for t in range(T)], axis=0)
    xc = jnp.concatenate([xbt, xrev], axis=-1)
    gx = jnp.dot(xc.reshape(T * R, 2 * F_IN), wih_ref[...],
                 preferred_element_type=jnp.float32) + bih_ref[...]
    gx = gx.reshape(T, R, 6 * H)
    whh = whh_ref[...]
    bhh = bhh_ref[...]
    h = jnp.zeros((R, 2 * H), jnp.float32)
    acc = jnp.zeros((R, 2 * H), jnp.float32)
    for t in range(T):
        gh = jnp.dot(h.astype(jnp.bfloat16), whh,
                     preferred_element_type=jnp.float32) + bhh
        gxt = gx[t]
        rz = _sigmoid(gxt[:, 0:256] + gh[:, 0:256])
        r = rz[:, 0:128]
        z = rz[:, 128:256]
        n = jnp.tanh(gxt[:, 256:384] + r * gh[:, 256:384])
        h = (1.0 - z) * n + z * h
        acc = acc + h
    out_ref[...] = acc * (1.0 / T)


def _gcn_kernel(a_ref, tm_ref, w1_ref, b1_ref, w2_ref, b2_ref,
                cw_ref, cb_ref, maskt_ref, out_ref):
    a_hi, a_lo = _split(a_ref[...])
    w1_hi, w1_lo = _split(w1_ref[...])
    w2_hi, w2_lo = _split(w2_ref[...])
    b1 = b1_ref[...]
    b2 = b2_ref[...]
    cw = cw_ref[...]
    cb = cb_ref[0, 0]
    maskt = maskt_ref[...]
    for b in range(B):
        t_hi, t_lo = _split(tm_ref[b])
        u1_hi, u1_lo = _split(_dot3(a_hi, a_lo, t_hi, t_lo))
        h1 = jnp.maximum(_dot3(u1_hi, u1_lo, w1_hi, w1_lo) + b1, 0.0)
        h1_hi, h1_lo = _split(h1)
        u2_hi, u2_lo = _split(_dot3(a_hi, a_lo, h1_hi, h1_lo))
        h2 = jnp.maximum(_dot3(u2_hi, u2_lo, w2_hi, w2_lo) + b2, 0.0)
        lg = jnp.dot(h2, cw, preferred_element_type=jnp.float32) + cb
        logits = jnp.transpose(lg)
        masked = jnp.where(maskt == 0, -1e9, logits)
        m = jnp.max(masked, axis=1, keepdims=True)
        sh = masked - m
        lse = jnp.log(jnp.sum(jnp.exp(sh), axis=1, keepdims=True))
        out_ref[b] = sh - lse


def _blkdiag(a, b):
    z = jnp.zeros_like(a)
    return jnp.concatenate(
        [jnp.concatenate([a, z], axis=1), jnp.concatenate([z, b], axis=1)], axis=0)


@jax.jit
def kernel(x, edges, masks, W_ih_f, W_hh_f, b_ih_f, b_hh_f,
           W_ih_b, W_hh_b, b_ih_b, b_hh_b,
           gcn1_W, gcn1_b, gcn2_W, gcn2_b, cls_W, cls_b):
    wih_f, wih_b = W_ih_f.T, W_ih_b.T
    whh_f, whh_b = W_hh_f.T, W_hh_b.T
    wih = jnp.concatenate(
        [_blkdiag(wih_f[:, i * H:(i + 1) * H], wih_b[:, i * H:(i + 1) * H])
         for i in range(3)], axis=1)
    whh = jnp.concatenate(
        [_blkdiag(whh_f[:, i * H:(i + 1) * H], whh_b[:, i * H:(i + 1) * H])
         for i in range(3)], axis=1)
    bih = jnp.concatenate(
        [jnp.concatenate([b_ih_f[i * H:(i + 1) * H], b_ih_b[i * H:(i + 1) * H]])
         for i in range(3)]).reshape(1, 6 * H)
    bhh = jnp.concatenate(
        [jnp.concatenate([b_hh_f[i * H:(i + 1) * H], b_hh_b[i * H:(i + 1) * H]])
         for i in range(3)]).reshape(1, 6 * H)

    xct = x.astype(jnp.bfloat16).reshape(B * N, T, F_IN).transpose(1, 0, 2)

    grid = (B * N) // R
    temporal = pl.pallas_call(
        _gru_kernel,
        grid=(grid,),
        in_specs=[
            pl.BlockSpec((T, R, F_IN), lambda i: (0, i, 0)),
            pl.BlockSpec((2 * F_IN, 6 * H), lambda i: (0, 0)),
            pl.BlockSpec((2 * H, 6 * H), lambda i: (0, 0)),
            pl.BlockSpec((1, 6 * H), lambda i: (0, 0)),
            pl.BlockSpec((1, 6 * H), lambda i: (0, 0)),
        ],
        out_specs=pl.BlockSpec((R, 2 * H), lambda i: (i, 0)),
        out_shape=jax.ShapeDtypeStruct((B * N, 2 * H), jnp.float32),
    )(xct, wih.astype(jnp.bfloat16), whh.astype(jnp.bfloat16), bih, bhh)

    tm = temporal.reshape(B, N, 2 * H)
    maskt = masks.T.astype(jnp.int32)

    preds = pl.pallas_call(
        _gcn_kernel,
        out_shape=jax.ShapeDtypeStruct((B, C, N), jnp.float32),
    )(edges, tm, gcn1_W, gcn1_b.reshape(1, 2 * H), gcn2_W,
      gcn2_b.reshape(1, 2 * H), cls_W, cls_b.reshape(1, 1), maskt)

    return preds


# speedup vs baseline: 1.3230x; 1.3230x over previous
import jax
import jax.numpy as jnp
from jax.experimental import pallas as pl

B = 2
N = 1000
T = 16
F_IN = 64
H = 64
C = 12
R = 400


def _sigmoid(v):
    return 0.5 * jnp.tanh(0.5 * v) + 0.5


def _gru_kernel(xct_ref, wih_ref, whh_ref, bih_ref, bhh_ref, out_ref):
    xbt = xct_ref[...]
    xrev = jnp.concatenate([xbt[T - 1 - t:T - t] for t in range(T)], axis=0)
    xc = jnp.concatenate([xbt, xrev], axis=-1)
    gx = jnp.dot(xc.reshape(T * R, 2 * F_IN), wih_ref[...],
                 preferred_element_type=jnp.float32) + bih_ref[...]
    gx = gx.reshape(T, R, 6 * H)
    whh = whh_ref[...]
    bhh = bhh_ref[...]
    h = jnp.zeros((R, 2 * H), jnp.float32)
    acc = jnp.zeros((R, 2 * H), jnp.float32)
    for t in range(T):
        gh = jnp.dot(h.astype(jnp.bfloat16), whh,
                     preferred_element_type=jnp.float32) + bhh
        gxt = gx[t]
        rz = _sigmoid(gxt[:, 0:256] + gh[:, 0:256])
        r = rz[:, 0:128]
        z = rz[:, 128:256]
        n = jnp.tanh(gxt[:, 256:384] + r * gh[:, 256:384])
        h = (1.0 - z) * n + z * h
        acc = acc + h
    out_ref[...] = acc * (1.0 / T)


def _gcn_kernel(a_ref, tm_ref, w1_ref, b1_ref, w2_ref, b2_ref,
                cw_ref, cb_ref, maskt_ref, out_ref):
    a = a_ref[...]
    w1 = w1_ref[...]
    w2 = w2_ref[...]
    b1 = b1_ref[...]
    b2 = b2_ref[...]
    cw = cw_ref[...]
    cb = cb_ref[0, 0]
    maskt = maskt_ref[...]
    for b in range(B):
        tm = tm_ref[b]
        u1 = jnp.dot(a, tm, preferred_element_type=jnp.float32)
        h1 = jnp.maximum(jnp.dot(u1, w1, preferred_element_type=jnp.float32) + b1, 0.0)
        u2 = jnp.dot(a, h1, preferred_element_type=jnp.float32)
        h2 = jnp.maximum(jnp.dot(u2, w2, preferred_element_type=jnp.float32) + b2, 0.0)
        lg = jnp.dot(h2, cw, preferred_element_type=jnp.float32) + cb
        logits = jnp.transpose(lg)
        masked = jnp.where(maskt == 0, -1e9, logits)
        m = jnp.max(masked, axis=1, keepdims=True)
        sh = masked - m
        lse = jnp.log(jnp.sum(jnp.exp(sh), axis=1, keepdims=True))
        out_ref[b] = sh - lse


def _blkdiag(a, b):
    z = jnp.zeros_like(a)
    return jnp.concatenate(
        [jnp.concatenate([a, z], axis=1), jnp.concatenate([z, b], axis=1)], axis=0)


@jax.jit
def kernel(x, edges, masks, W_ih_f, W_hh_f, b_ih_f, b_hh_f,
           W_ih_b, W_hh_b, b_ih_b, b_hh_b,
           gcn1_W, gcn1_b, gcn2_W, gcn2_b, cls_W, cls_b):
    wih_f, wih_b = W_ih_f.T, W_ih_b.T
    whh_f, whh_b = W_hh_f.T, W_hh_b.T
    wih = jnp.concatenate(
        [_blkdiag(wih_f[:, i * H:(i + 1) * H], wih_b[:, i * H:(i + 1) * H])
         for i in range(3)], axis=1)
    whh = jnp.concatenate(
        [_blkdiag(whh_f[:, i * H:(i + 1) * H], whh_b[:, i * H:(i + 1) * H])
         for i in range(3)], axis=1)
    bih = jnp.concatenate(
        [jnp.concatenate([b_ih_f[i * H:(i + 1) * H], b_ih_b[i * H:(i + 1) * H]])
         for i in range(3)]).reshape(1, 6 * H)
    bhh = jnp.concatenate(
        [jnp.concatenate([b_hh_f[i * H:(i + 1) * H], b_hh_b[i * H:(i + 1) * H]])
         for i in range(3)]).reshape(1, 6 * H)

    xct = x.astype(jnp.bfloat16).reshape(B * N, T, F_IN).transpose(1, 0, 2)

    grid = (B * N) // R
    temporal = pl.pallas_call(
        _gru_kernel,
        grid=(grid,),
        in_specs=[
            pl.BlockSpec((T, R, F_IN), lambda i: (0, i, 0)),
            pl.BlockSpec((2 * F_IN, 6 * H), lambda i: (0, 0)),
            pl.BlockSpec((2 * H, 6 * H), lambda i: (0, 0)),
            pl.BlockSpec((1, 6 * H), lambda i: (0, 0)),
            pl.BlockSpec((1, 6 * H), lambda i: (0, 0)),
        ],
        out_specs=pl.BlockSpec((R, 2 * H), lambda i: (i, 0)),
        out_shape=jax.ShapeDtypeStruct((B * N, 2 * H), jnp.float32),
    )(xct, wih.astype(jnp.bfloat16), whh.astype(jnp.bfloat16), bih, bhh)

    tm = temporal.reshape(B, N, 2 * H)
    maskt = masks.T.astype(jnp.int32)

    preds = pl.pallas_call(
        _gcn_kernel,
        out_shape=jax.ShapeDtypeStruct((B, C, N), jnp.float32),
    )(edges, tm, gcn1_W, gcn1_b.reshape(1, 2 * H), gcn2_W,
      gcn2_b.reshape(1, 2 * H), cls_W, cls_b.reshape(1, 1), maskt)

    return preds
